# Initial kernel scaffold; baseline (speedup 1.0000x reference)
import jax
import jax.numpy as jnp
from jax.experimental import pallas as pl


def kernel(x, edge_index, W1, b1, Wc, W2, b2):
    raise NotImplementedError("write your pallas kernel here")


# speedup vs baseline: 13.0902x; 13.0902x over previous
import functools
import math

import jax
import jax.numpy as jnp
from jax import lax
from jax.experimental import pallas as pl
from jax.experimental.pallas import tpu as pltpu
from jax.experimental.pallas import tpu_sc as plsc

N = 10000
E = 320000
F_IN = 128
HID = 64
CLS = 64
K = 16
ALPHA = 0.1
LAMDA = 0.5

NC = 2
NS = 16
NW = NC * NS
CHUNK = 128
EP = E + N
NCH = -(-EP // (NW * CHUNK))
E_PAD = NW * CHUNK * NCH
N_PAD = 10112
ROWS_PER_SUB = N_PAD // NS
_WB = [128, 128, 128, 128, ROWS_PER_SUB - 512]


def _sc_spmm_body(tbl_hbm, src_hbm, dst_hbm, out_hbm, agg, src_v, dst_v, buf,
                  gsem):
    c = lax.axis_index("c")
    s = lax.axis_index("s")
    w = c * NS + s

    def _zb(i, _):
        for j in range(HID // 16):
            buf[i, pl.ds(j * 16, 16)] = jnp.zeros((16,), jnp.float32)
        return 0

    lax.fori_loop(0, CHUNK, _zb, 0)
    base = pl.multiple_of(s * ROWS_PER_SUB, 8)
    off = 0
    for sz in _WB:
        pltpu.sync_copy(buf.at[pl.ds(0, sz)], agg.at[pl.ds(base + off, sz)])
        off += sz
    plsc.subcore_barrier()

    pltpu.sync_copy(src_hbm.at[w], src_v)
    pltpu.sync_copy(dst_hbm.at[w], dst_v)

    def _step(j, _):
        pltpu.async_copy(tbl_hbm.at[src_v.at[j]], buf, gsem).wait()
        pltpu.sync_copy(buf, agg.at[dst_v.at[j]], add=True)
        return 0

    lax.fori_loop(0, NCH, _step, 0)
    plsc.subcore_barrier()

    off = 0
    for sz in _WB:
        pltpu.sync_copy(agg.at[pl.ds(base + off, sz)], buf.at[pl.ds(0, sz)])
        pltpu.sync_copy(buf.at[pl.ds(0, sz)],
                        out_hbm.at[c, pl.ds(base + off, sz)])
        off += sz


@functools.cache
def _spmm_kernel():
    return pl.kernel(
        _sc_spmm_body,
        out_type=jax.ShapeDtypeStruct((NC, N_PAD, HID), jnp.float32),
        mesh=plsc.VectorSubcoreMesh(core_axis_name="c", subcore_axis_name="s",
                                    num_cores=NC, num_subcores=NS),
        scratch_types=[
            pltpu.VMEM_SHARED((N_PAD, HID), jnp.float32),
            pltpu.VMEM((NCH, CHUNK), jnp.int32),
            pltpu.VMEM((NCH, CHUNK), jnp.int32),
            pltpu.VMEM((CHUNK, HID), jnp.float32),
            pltpu.SemaphoreType.DMA,
        ],
        compiler_params=pltpu.CompilerParams(use_tc_tiling_on_sc=False),
    )


def _spmm(tbl, src, dst):
    return _spmm_kernel()(tbl, src, dst)


def _tc_pre_body(x_ref, w1_ref, b1_ref, degp_ref, h0_ref, dinv_ref, g0_ref):
    deg = degp_ref[0] + degp_ref[1]
    dinv = jnp.where(deg > 0, lax.rsqrt(jnp.maximum(deg, 1e-12)), 0.0)
    h0 = jnp.maximum(
        jnp.dot(x_ref[...], w1_ref[...], preferred_element_type=jnp.float32)
        + b1_ref[...], 0.0)
    h0_ref[...] = h0
    dinv_ref[...] = dinv
    g0_ref[...] = dinv * h0


def _tc_pre(x_pad, W1, b1, degp):
    sds = jax.ShapeDtypeStruct((N_PAD, HID), jnp.float32)
    return pl.pallas_call(
        _tc_pre_body, out_shape=(sds, sds, sds))(x_pad, W1, b1, degp)


def _tc_step_body(pp_ref, h0_ref, dinv_ref, m_ref, g_ref):
    dinv = dinv_ref[...]
    support = (1.0 - ALPHA) * (dinv * (pp_ref[0] + pp_ref[1])) \
        + ALPHA * h0_ref[...]
    g_ref[...] = dinv * jnp.dot(support, m_ref[...],
                                preferred_element_type=jnp.float32)


def _tc_step(pp, h0, dinv, M):
    return pl.pallas_call(
        _tc_step_body,
        out_shape=jax.ShapeDtypeStruct((N_PAD, HID), jnp.float32),
    )(pp, h0, dinv, M)


def _tc_fin_body(pp_ref, h0_ref, dinv_ref, m_ref, w2_ref, b2_ref, out_ref):
    dinv = dinv_ref[...]
    support = (1.0 - ALPHA) * (dinv * (pp_ref[0] + pp_ref[1])) \
        + ALPHA * h0_ref[...]
    h = jnp.dot(support, m_ref[...], preferred_element_type=jnp.float32)
    logits = jnp.dot(h, w2_ref[...], preferred_element_type=jnp.float32) \
        + b2_ref[...]
    mx = jnp.max(logits, axis=1, keepdims=True)
    lse = jnp.log(jnp.sum(jnp.exp(logits - mx), axis=1, keepdims=True)) + mx
    out_ref[...] = logits - lse


def _tc_fin(pp, h0, dinv, M, W2, b2):
    return pl.pallas_call(
        _tc_fin_body,
        out_shape=jax.ShapeDtypeStruct((N_PAD, CLS), jnp.float32),
    )(pp, h0, dinv, M, W2, b2)


def kernel(x, edge_index, W1, b1, Wc, W2, b2):
    loop = jnp.arange(N, dtype=jnp.int32)
    src = jnp.concatenate(
        [edge_index[0].astype(jnp.int32), loop,
         jnp.full((E_PAD - EP,), N, jnp.int32)]).reshape(NW, NCH, CHUNK)
    dst = jnp.concatenate(
        [edge_index[1].astype(jnp.int32), loop,
         jnp.zeros((E_PAD - EP,), jnp.int32)]).reshape(NW, NCH, CHUNK)

    x_pad = jnp.concatenate(
        [x, jnp.zeros((N_PAD - N, F_IN), jnp.float32)])
    ones_tbl = jnp.concatenate(
        [jnp.ones((N, HID), jnp.float32),
         jnp.zeros((N_PAD - N, HID), jnp.float32)])
    b1r = jnp.broadcast_to(b1[None, :], (1, HID))
    b2r = jnp.broadcast_to(b2[None, :], (1, CLS))

    degp = _spmm(ones_tbl, src, dst)
    h0, dinv, g = _tc_pre(x_pad, W1, b1r, degp)

    eye = jnp.eye(HID, dtype=jnp.float32)
    for l in range(1, K + 1):
        beta = math.log(LAMDA / l + 1.0)
        M = (1.0 - beta) * eye + beta * Wc
        pp = _spmm(g, src, dst)
        if l < K:
            g = _tc_step(pp, h0, dinv, M)
        else:
            out = _tc_fin(pp, h0, dinv, M, W2, b2r)
    return out[:N]
